# baseline (speedup 1.0000x reference)
import functools

import jax
import jax.numpy as jnp
from jax import lax
from jax.experimental import pallas as pl
from jax.experimental.pallas import tpu as pltpu
from jax.experimental.pallas import tpu_sc as plsc

_B, _H, _W, _D = 2, 224, 224, 256
_NC, _NS = 2, 16
_NW = _NC * _NS
_HPW = _H // _NW
_HALF = _W // 2
_L = 16

_mesh = plsc.VectorSubcoreMesh(
    core_axis_name="c", subcore_axis_name="s", num_cores=_NC, num_subcores=_NS
)


@functools.partial(
    pl.kernel,
    mesh=_mesh,
    out_type=jax.ShapeDtypeStruct((_B, _H * _W, _D), jnp.float32),
    scratch_types=[
        pltpu.VMEM((_W, _D), jnp.float32),
        pltpu.VMEM((16, _D), jnp.float32),
        pltpu.VMEM((2, _HALF, _D), jnp.float32),
        pltpu.SemaphoreType.DMA,
        pltpu.SemaphoreType.DMA,
    ],
)
def _sc_pos_enc(row_hbm, col_hbm, out_hbm, col_buf, row_buf, obuf, sem0, sem1):
    wid = lax.axis_index("s") * _NC + lax.axis_index("c")
    h0 = wid * _HPW

    base8 = (h0 // 8) * 8
    roff = h0 - base8
    pltpu.sync_copy(col_hbm.at[pl.ds(0, _W)], col_buf)
    pltpu.sync_copy(row_hbm.at[pl.ds(base8, 16)], row_buf)

    sems = (sem0, sem1)
    pending = [None, None]
    for hl in range(_HPW):
        row_vecs = [
            row_buf[roff + hl, pl.ds(j * _L, _L)] for j in range(_D // _L)
        ]
        for half in range(2):
            u = hl * 2 + half
            ph = u % 2
            if pending[ph] is not None:
                for c in pending[ph]:
                    c.wait()

            @pl.loop(0, _HALF)
            def _(w, half=half, ph=ph, row_vecs=row_vecs):
                for j in range(_D // _L):
                    obuf[ph, w, pl.ds(j * _L, _L)] = (
                        col_buf[half * _HALF + w, pl.ds(j * _L, _L)]
                        + row_vecs[j]
                    )

            base = (h0 + hl) * _W + half * _HALF
            c0 = pltpu.async_copy(
                obuf.at[ph], out_hbm.at[0, pl.ds(base, _HALF)], sems[ph]
            )
            c1 = pltpu.async_copy(
                obuf.at[ph], out_hbm.at[1, pl.ds(base, _HALF)], sems[ph]
            )
            pending[ph] = (c0, c1)

    for ph in range(2):
        for c in pending[ph]:
            c.wait()


def kernel(batch_size, height, width, row_embed, col_embed):
    return _sc_pos_enc(row_embed, col_embed)


# speedup vs baseline: 1.6307x; 1.6307x over previous
import functools

import jax
import jax.numpy as jnp
from jax import lax
from jax.experimental import pallas as pl
from jax.experimental.pallas import tpu as pltpu
from jax.experimental.pallas import tpu_sc as plsc

_B, _H, _W, _D = 2, 224, 224, 256
_NC, _NS = 2, 16
_NW = _NC * _NS
_HPW = _H // _NW
_HALF = _W // 2
_L = 16

_mesh = plsc.VectorSubcoreMesh(
    core_axis_name="c", subcore_axis_name="s", num_cores=_NC, num_subcores=_NS
)


@functools.partial(
    pl.kernel,
    mesh=_mesh,
    out_type=jax.ShapeDtypeStruct((_B, _H * _W, _D), jnp.float32),
    scratch_types=[
        pltpu.VMEM((_W, _D), jnp.float32),
        pltpu.VMEM((16, _D), jnp.float32),
        pltpu.VMEM((2, _HALF, _D), jnp.float32),
        pltpu.SemaphoreType.DMA,
        pltpu.SemaphoreType.DMA,
    ],
)
def _sc_pos_enc(row_hbm, col_hbm, out_hbm, col_buf, row_buf, obuf, sem0, sem1):
    wid = lax.axis_index("s") * _NC + lax.axis_index("c")
    h0 = wid * _HPW

    base8 = (h0 // 8) * 8
    roff = h0 - base8
    pltpu.sync_copy(col_hbm.at[pl.ds(0, _W)], col_buf)
    pltpu.sync_copy(row_hbm.at[pl.ds(base8, 16)], row_buf)

    sems = (sem0, sem1)
    pending = [None, None]
    for hl in range(_HPW):
        row_vecs = [
            row_buf[roff + hl, pl.ds(j * _L, _L)] for j in range(_D // _L)
        ]
        for half in range(2):
            u = hl * 2 + half
            ph = u % 2
            if pending[ph] is not None:
                for c in pending[ph]:
                    c.wait()

            @plsc.parallel_loop(0, _HALF, unroll=4)
            def _(w, half=half, ph=ph, row_vecs=row_vecs):
                for j in range(_D // _L):
                    obuf[ph, w, pl.ds(j * _L, _L)] = (
                        col_buf[half * _HALF + w, pl.ds(j * _L, _L)]
                        + row_vecs[j]
                    )

            base = (h0 + hl) * _W + half * _HALF
            c0 = pltpu.async_copy(
                obuf.at[ph], out_hbm.at[0, pl.ds(base, _HALF)], sems[ph]
            )
            c1 = pltpu.async_copy(
                obuf.at[ph], out_hbm.at[1, pl.ds(base, _HALF)], sems[ph]
            )
            pending[ph] = (c0, c1)

    for ph in range(2):
        for c in pending[ph]:
            c.wait()


def kernel(batch_size, height, width, row_embed, col_embed):
    return _sc_pos_enc(row_embed, col_embed)


# speedup vs baseline: 1.6352x; 1.0028x over previous
import functools

import jax
import jax.numpy as jnp
from jax import lax
from jax.experimental import pallas as pl
from jax.experimental.pallas import tpu as pltpu
from jax.experimental.pallas import tpu_sc as plsc

_B, _H, _W, _D = 2, 224, 224, 256
_NC, _NS = 2, 16
_NW = _NC * _NS
_HPW = _H // _NW
_HALF = _W // 2
_L = 16

_mesh = plsc.VectorSubcoreMesh(
    core_axis_name="c", subcore_axis_name="s", num_cores=_NC, num_subcores=_NS
)


@functools.partial(
    pl.kernel,
    mesh=_mesh,
    out_type=jax.ShapeDtypeStruct((_B, _H * _W, _D), jnp.float32),
    scratch_types=[
        pltpu.VMEM((_W, _D), jnp.float32),
        pltpu.VMEM((16, _D), jnp.float32),
        pltpu.VMEM((2, _HALF, _D), jnp.float32),
        pltpu.SemaphoreType.DMA,
        pltpu.SemaphoreType.DMA,
    ],
)
def _sc_pos_enc(row_hbm, col_hbm, out_hbm, col_buf, row_buf, obuf, sem0, sem1):
    wid = lax.axis_index("s") * _NC + lax.axis_index("c")
    h0 = wid * _HPW

    base8 = (h0 // 8) * 8
    roff = h0 - base8
    pltpu.sync_copy(col_hbm.at[pl.ds(0, _W)], col_buf)
    pltpu.sync_copy(row_hbm.at[pl.ds(base8, 16)], row_buf)

    sems = (sem0, sem1)
    pending = [None, None]
    for hl in range(_HPW):
        row_vecs = [
            row_buf[roff + hl, pl.ds(j * _L, _L)] for j in range(_D // _L)
        ]
        for half in range(2):
            u = hl * 2 + half
            ph = u % 2
            if pending[ph] is not None:
                for c in pending[ph]:
                    c.wait()

            @plsc.parallel_loop(0, _HALF, unroll=8)
            def _(w, half=half, ph=ph, row_vecs=row_vecs):
                for j in range(_D // _L):
                    obuf[ph, w, pl.ds(j * _L, _L)] = (
                        col_buf[half * _HALF + w, pl.ds(j * _L, _L)]
                        + row_vecs[j]
                    )

            base = (h0 + hl) * _W + half * _HALF
            c0 = pltpu.async_copy(
                obuf.at[ph], out_hbm.at[0, pl.ds(base, _HALF)], sems[ph]
            )
            c1 = pltpu.async_copy(
                obuf.at[ph], out_hbm.at[1, pl.ds(base, _HALF)], sems[ph]
            )
            pending[ph] = (c0, c1)

    for ph in range(2):
        for c in pending[ph]:
            c.wait()


def kernel(batch_size, height, width, row_embed, col_embed):
    return _sc_pos_enc(row_embed, col_embed)
